# baseline (speedup 1.0000x reference)
import functools

import jax
import jax.numpy as jnp
from jax import lax
from jax.experimental import pallas as pl
from jax.experimental.pallas import tpu as pltpu
from jax.experimental.pallas import tpu_sc as plsc

B = 16384
D = 64
DP = 128
V = 1000000
K = 20
NC = 2
NS = 16
L = 16
NW = NC * NS
BPW = B // NW
BC = 16
BCK = BC * K
NCH = BPW // BC
NPAIR = NCH // 2

NBLK = V // DP
NB_MAIN = (NBLK // NW) & ~1
NPAIR_T = NB_MAIN // 2
NB_EXTRA = NBLK - NB_MAIN * NW
VTAIL = V - NBLK * DP


def _sc_transpose(tin_t, tout_t, tail_in, tail_out):
  mesh = plsc.VectorSubcoreMesh(core_axis_name="c", subcore_axis_name="s")
  f32 = jnp.float32
  i32 = jnp.int32

  @functools.partial(
      pl.kernel,
      out_type=(
          jax.ShapeDtypeStruct((V, DP), f32),
          jax.ShapeDtypeStruct((V, DP), f32),
      ),
      mesh=mesh,
      compiler_params=pltpu.CompilerParams(
          needs_layout_passes=False, use_tc_tiling_on_sc=True),
      scratch_types=[
          pltpu.VMEM((D, DP), f32),
          pltpu.VMEM((D, DP), f32),
          pltpu.VMEM((DP, DP), f32),
          pltpu.VMEM((DP, DP), f32),
          pltpu.VMEM((VTAIL, D), f32),
          pltpu.SemaphoreType.DMA,
          pltpu.SemaphoreType.DMA,
          pltpu.SemaphoreType.DMA,
          pltpu.SemaphoreType.DMA,
      ],
  )
  def tk(tinT, toutT, tl_in, tl_out, o_in, o_out,
         ibA, ibB, obA, obB, tbuf, semIA, semIB, semOA, semOB):
    wid = lax.axis_index("s") * NC + lax.axis_index("c")
    lane = lax.iota(i32, L)
    dlane = [lane + dc * L for dc in range(D // L)]

    def transpose(ib, ob):
      def vbody(v4, _):
        for uv in range(4):
          v = v4 * 4 + uv
          vs = jnp.broadcast_to(v, (L,)).astype(i32)
          for dc in range(D // L):
            vec = plsc.load_gather(ib, [dlane[dc], vs])
            plsc.store_scatter(ob, [vs, dlane[dc]], vec)
        return 0
      lax.fori_loop(0, DP // 4, vbody, 0)

    def run_table(tT, O, tail):
      def fire_in(i, ib, sem):
        blk = i * NW + wid
        pltpu.async_copy(tT.at[:, pl.ds(blk * DP, DP)], ib, sem)

      def drain_in(ib, sem):
        pltpu.make_async_copy(tT.at[:, pl.ds(0, DP)], ib, sem).wait()

      def fire_out(i, ob, sem):
        blk = i * NW + wid
        pltpu.async_copy(ob, O.at[pl.ds(blk * DP, DP), :], sem)

      def drain_out(ob, sem):
        pltpu.make_async_copy(ob, O.at[pl.ds(0, DP), :], sem).wait()

      fire_in(0, ibA, semIA)

      def pbody(p, _):
        i0 = 2 * p
        fire_in(i0 + 1, ibB, semIB)
        drain_in(ibA, semIA)

        @pl.when(p > 0)
        def _():
          drain_out(obA, semOA)

        transpose(ibA, obA)
        fire_out(i0, obA, semOA)

        @pl.when(p < NPAIR_T - 1)
        def _():
          fire_in(i0 + 2, ibA, semIA)

        drain_in(ibB, semIB)

        @pl.when(p > 0)
        def _():
          drain_out(obB, semOB)

        transpose(ibB, obB)
        fire_out(i0 + 1, obB, semOB)
        return 0

      lax.fori_loop(0, NPAIR_T, pbody, 0)
      drain_out(obA, semOA)
      drain_out(obB, semOB)

      @pl.when(wid < NB_EXTRA)
      def _():
        blk = NB_MAIN * NW + wid
        pltpu.sync_copy(tT.at[:, pl.ds(blk * DP, DP)], ibA)
        transpose(ibA, obA)
        pltpu.sync_copy(obA, O.at[pl.ds(blk * DP, DP), :])

      @pl.when(wid == NB_EXTRA)
      def _():
        pltpu.sync_copy(tail, tbuf)

        def tbody(r, _):
          for dc in range(D // L):
            rs = jnp.broadcast_to(r, (L,)).astype(i32)
            vec = plsc.load_gather(tbuf, [rs, dlane[dc]])
            plsc.store_scatter(obB, [rs, dlane[dc]], vec)
          return 0

        lax.fori_loop(0, VTAIL, tbody, 0)
        pltpu.sync_copy(obB.at[pl.ds(0, VTAIL), :],
                        O.at[pl.ds(NBLK * DP, VTAIL), :])

    run_table(tinT, o_in, tl_in)
    run_table(toutT, o_out, tl_out)

  return tk(tin_t, tout_t, tail_in, tail_out)


def _sc_scores(target_w, context_w, neg_w_flat, emb_input_p, emb_output_p):
  mesh = plsc.VectorSubcoreMesh(core_axis_name="c", subcore_axis_name="s")
  f32 = jnp.float32
  i32 = jnp.int32

  @functools.partial(
      pl.kernel,
      out_type=(
          jax.ShapeDtypeStruct((B,), f32),
          jax.ShapeDtypeStruct((B * K,), f32),
      ),
      mesh=mesh,
      compiler_params=pltpu.CompilerParams(
          needs_layout_passes=False, use_tc_tiling_on_sc=True),
      scratch_types=[
          pltpu.VMEM((BPW,), i32),
          pltpu.VMEM((BPW,), i32),
          pltpu.VMEM((BPW * K,), i32),
          pltpu.VMEM((2, BC, DP), f32),
          pltpu.VMEM((2, BC, DP), f32),
          pltpu.VMEM((2, BCK, DP), f32),
          pltpu.VMEM((BPW,), f32),
          pltpu.VMEM((BPW * K,), f32),
          pltpu.SemaphoreType.DMA,
          pltpu.SemaphoreType.DMA,
      ],
  )
  def sc_kernel(tgt_hbm, ctx_hbm, negi_hbm, tin_hbm, tout_hbm,
                pos_hbm, nego_hbm,
                tidx, cidx, nidx, trows, crows, nrows, posb, negb,
                semA, semB):
    wid = lax.axis_index("s") * NC + lax.axis_index("c")
    base = wid * BPW
    lane = lax.iota(i32, L)

    pltpu.sync_copy(tgt_hbm.at[pl.ds(base, BPW)], tidx)
    pltpu.sync_copy(ctx_hbm.at[pl.ds(base, BPW)], cidx)
    pltpu.sync_copy(negi_hbm.at[pl.ds(base * K, BPW * K)], nidx)

    def fire(c, slot, sem):
      pltpu.async_copy(tin_hbm.at[tidx.at[pl.ds(c * BC, BC)]],
                       trows.at[slot], sem)
      pltpu.async_copy(tout_hbm.at[cidx.at[pl.ds(c * BC, BC)]],
                       crows.at[slot], sem)
      for j in range(0, BCK, 64):
        pltpu.async_copy(
            tout_hbm.at[nidx.at[pl.ds(c * BCK + j, 64)]],
            nrows.at[slot].at[pl.ds(j, 64), :], sem)

    def drain(slot, sem):
      pltpu.make_async_copy(tin_hbm.at[tidx.at[pl.ds(0, BC)]],
                            trows.at[slot], sem).wait()
      pltpu.make_async_copy(tout_hbm.at[cidx.at[pl.ds(0, BC)]],
                            crows.at[slot], sem).wait()
      for j in range(0, BCK, 64):
        pltpu.make_async_copy(
            tout_hbm.at[nidx.at[pl.ds(j, 64)]],
            nrows.at[slot].at[pl.ds(j, 64), :], sem).wait()

    def compute(c, slot):
      tro, cro, nro = trows.at[slot], crows.at[slot], nrows.at[slot]
      rows = lane
      rowsK = lane * K
      zf = jnp.zeros((L,), f32)

      def jbody(j, carry):
        dv = carry[0]
        accp = carry[1]
        accn = list(carry[2:])
        for u in range(4):
          dvu = dv + u
          t = plsc.load_gather(tro, [rows, dvu])
          cv = plsc.load_gather(cro, [rows, dvu])
          accp = accp + t * cv
          for k in range(K):
            accn[k] = accn[k] + t * plsc.load_gather(nro, [rowsK + k, dvu])
        return (dv + 4, accp, *accn)

      out = lax.fori_loop(0, D // 4, jbody,
                          (jnp.zeros((L,), i32), zf, *([zf] * K)))
      accp = out[1]
      accn = out[2:]
      posb[pl.ds(c * BC, L)] = accp
      for k in range(K):
        negb[pl.ds(c * BCK + k * BC, L)] = accn[k]

    fire(0, 0, semA)

    def pair(p, _):
      ca = 2 * p
      fire(ca + 1, 1, semB)
      drain(0, semA)
      compute(ca, 0)

      @pl.when(p < NPAIR - 1)
      def _():
        fire(ca + 2, 0, semA)

      drain(1, semB)
      compute(ca + 1, 1)
      return 0

    lax.fori_loop(0, NPAIR, pair, 0)
    pltpu.sync_copy(posb, pos_hbm.at[pl.ds(base, BPW)])
    pltpu.sync_copy(negb, nego_hbm.at[pl.ds(base * K, BPW * K)])

  return sc_kernel(target_w, context_w, neg_w_flat, emb_input_p, emb_output_p)


def _tc_loss(pos2, neg2):
  f32 = jnp.float32

  def tc_body(pos_ref, neg_ref, out_ref):
    p = pos_ref[...]
    n = neg_ref[...]

    def sp(x):
      return jnp.maximum(x, 0.0) + jnp.log1p(jnp.exp(-jnp.abs(x)))

    out_ref[0, 0] = (jnp.sum(sp(-p)) + jnp.sum(sp(n))) / B

  return pl.pallas_call(
      tc_body,
      out_shape=jax.ShapeDtypeStruct((1, 1), f32),
      out_specs=pl.BlockSpec(memory_space=pltpu.SMEM),
  )(pos2, neg2)


def kernel(target_w, context_w, neg_w, emb_input, emb_output):
  neg_w_flat = neg_w.astype(jnp.int32).reshape(B * K)
  tail_in = emb_input[NBLK * DP:, :]
  tail_out = emb_output[NBLK * DP:, :]
  emb_input_p, emb_output_p = _sc_transpose(
      emb_input.T, emb_output.T, tail_in, tail_out)
  pos, negs = _sc_scores(target_w.astype(jnp.int32),
                         context_w.astype(jnp.int32),
                         neg_w_flat, emb_input_p, emb_output_p)
  loss = _tc_loss(pos.reshape(B // 128, 128), negs.reshape(B * K // 128, 128))
  return loss[0, 0]


# speedup vs baseline: 1.6916x; 1.6916x over previous
import functools

import jax
import jax.numpy as jnp
from jax import lax
from jax.experimental import pallas as pl
from jax.experimental.pallas import tpu as pltpu
from jax.experimental.pallas import tpu_sc as plsc

B = 16384
D = 64
DP = 128
V = 1000000
K = 20
NC = 2
NS = 16
L = 16
NW = NC * NS
BPW = B // NW
BC = 16
BCK = BC * K
NCH = BPW // BC
NPAIR = NCH // 2

NBLK = V // DP
NB_MAIN = (NBLK // NW) & ~1
NPAIR_T = NB_MAIN // 2
NB_EXTRA = NBLK - NB_MAIN * NW
VTAIL = V - NBLK * DP


def _sc_transpose(tin_t, tout_t, tail_in, tail_out):
  mesh = plsc.VectorSubcoreMesh(core_axis_name="c", subcore_axis_name="s")
  f32 = jnp.float32
  i32 = jnp.int32

  @functools.partial(
      pl.kernel,
      out_type=(
          jax.ShapeDtypeStruct((V, DP), f32),
          jax.ShapeDtypeStruct((V, DP), f32),
      ),
      mesh=mesh,
      compiler_params=pltpu.CompilerParams(
          needs_layout_passes=False, use_tc_tiling_on_sc=True),
      scratch_types=[
          pltpu.VMEM((D, DP), f32),
          pltpu.VMEM((D, DP), f32),
          pltpu.VMEM((DP, DP), f32),
          pltpu.VMEM((DP, DP), f32),
          pltpu.VMEM((VTAIL, D), f32),
          pltpu.SemaphoreType.DMA,
          pltpu.SemaphoreType.DMA,
          pltpu.SemaphoreType.DMA,
          pltpu.SemaphoreType.DMA,
      ],
  )
  def tk(tinT, toutT, tl_in, tl_out, o_in, o_out,
         ibA, ibB, obA, obB, tbuf, semIA, semIB, semOA, semOB):
    wid = lax.axis_index("s") * NC + lax.axis_index("c")
    lane = lax.iota(i32, L)
    dlane = [lane + dc * L for dc in range(D // L)]

    def transpose(ib, ob):
      @plsc.parallel_loop(0, DP, unroll=4)
      def _(v):
        vs = jnp.broadcast_to(v, (L,)).astype(i32)
        for dc in range(D // L):
          vec = plsc.load_gather(ib, [dlane[dc], vs])
          plsc.store_scatter(ob, [vs, dlane[dc]], vec)

    def run_table(tT, O, tail):
      def fire_in(i, ib, sem):
        blk = i * NW + wid
        pltpu.async_copy(tT.at[:, pl.ds(blk * DP, DP)], ib, sem)

      def drain_in(ib, sem):
        pltpu.make_async_copy(tT.at[:, pl.ds(0, DP)], ib, sem).wait()

      def fire_out(i, ob, sem):
        blk = i * NW + wid
        pltpu.async_copy(ob, O.at[pl.ds(blk * DP, DP), :], sem)

      def drain_out(ob, sem):
        pltpu.make_async_copy(ob, O.at[pl.ds(0, DP), :], sem).wait()

      fire_in(0, ibA, semIA)

      def pbody(p, _):
        i0 = 2 * p
        fire_in(i0 + 1, ibB, semIB)
        drain_in(ibA, semIA)

        @pl.when(p > 0)
        def _():
          drain_out(obA, semOA)

        transpose(ibA, obA)
        fire_out(i0, obA, semOA)

        @pl.when(p < NPAIR_T - 1)
        def _():
          fire_in(i0 + 2, ibA, semIA)

        drain_in(ibB, semIB)

        @pl.when(p > 0)
        def _():
          drain_out(obB, semOB)

        transpose(ibB, obB)
        fire_out(i0 + 1, obB, semOB)
        return 0

      lax.fori_loop(0, NPAIR_T, pbody, 0)
      drain_out(obA, semOA)
      drain_out(obB, semOB)

      @pl.when(wid < NB_EXTRA)
      def _():
        blk = NB_MAIN * NW + wid
        pltpu.sync_copy(tT.at[:, pl.ds(blk * DP, DP)], ibA)
        transpose(ibA, obA)
        pltpu.sync_copy(obA, O.at[pl.ds(blk * DP, DP), :])

      @pl.when(wid == NB_EXTRA)
      def _():
        pltpu.sync_copy(tail, tbuf)

        @plsc.parallel_loop(0, VTAIL, unroll=4)
        def _(r):
          rs = jnp.broadcast_to(r, (L,)).astype(i32)
          for dc in range(D // L):
            vec = plsc.load_gather(tbuf, [rs, dlane[dc]])
            plsc.store_scatter(obB, [rs, dlane[dc]], vec)
        pltpu.sync_copy(obB.at[pl.ds(0, VTAIL), :],
                        O.at[pl.ds(NBLK * DP, VTAIL), :])

    run_table(tinT, o_in, tl_in)
    run_table(toutT, o_out, tl_out)

  return tk(tin_t, tout_t, tail_in, tail_out)


def _sc_scores(target_w, context_w, neg_w_flat, emb_input_p, emb_output_p):
  mesh = plsc.VectorSubcoreMesh(core_axis_name="c", subcore_axis_name="s")
  f32 = jnp.float32
  i32 = jnp.int32

  @functools.partial(
      pl.kernel,
      out_type=(
          jax.ShapeDtypeStruct((B,), f32),
          jax.ShapeDtypeStruct((B * K,), f32),
      ),
      mesh=mesh,
      compiler_params=pltpu.CompilerParams(
          needs_layout_passes=False, use_tc_tiling_on_sc=True),
      scratch_types=[
          pltpu.VMEM((BPW,), i32),
          pltpu.VMEM((BPW,), i32),
          pltpu.VMEM((BPW * K,), i32),
          pltpu.VMEM((2, BC, DP), f32),
          pltpu.VMEM((2, BC, DP), f32),
          pltpu.VMEM((2, BCK, DP), f32),
          pltpu.VMEM((BPW,), f32),
          pltpu.VMEM((BPW * K,), f32),
          pltpu.SemaphoreType.DMA,
          pltpu.SemaphoreType.DMA,
      ],
  )
  def sc_kernel(tgt_hbm, ctx_hbm, negi_hbm, tin_hbm, tout_hbm,
                pos_hbm, nego_hbm,
                tidx, cidx, nidx, trows, crows, nrows, posb, negb,
                semA, semB):
    wid = lax.axis_index("s") * NC + lax.axis_index("c")
    base = wid * BPW
    lane = lax.iota(i32, L)

    pltpu.sync_copy(tgt_hbm.at[pl.ds(base, BPW)], tidx)
    pltpu.sync_copy(ctx_hbm.at[pl.ds(base, BPW)], cidx)
    pltpu.sync_copy(negi_hbm.at[pl.ds(base * K, BPW * K)], nidx)

    def fire(c, slot, sem):
      pltpu.async_copy(tin_hbm.at[tidx.at[pl.ds(c * BC, BC)]],
                       trows.at[slot], sem)
      pltpu.async_copy(tout_hbm.at[cidx.at[pl.ds(c * BC, BC)]],
                       crows.at[slot], sem)
      for j in range(0, BCK, 64):
        pltpu.async_copy(
            tout_hbm.at[nidx.at[pl.ds(c * BCK + j, 64)]],
            nrows.at[slot].at[pl.ds(j, 64), :], sem)

    def drain(slot, sem):
      pltpu.make_async_copy(tin_hbm.at[tidx.at[pl.ds(0, BC)]],
                            trows.at[slot], sem).wait()
      pltpu.make_async_copy(tout_hbm.at[cidx.at[pl.ds(0, BC)]],
                            crows.at[slot], sem).wait()
      for j in range(0, BCK, 64):
        pltpu.make_async_copy(
            tout_hbm.at[nidx.at[pl.ds(j, 64)]],
            nrows.at[slot].at[pl.ds(j, 64), :], sem).wait()

    def compute(c, slot):
      tro, cro, nro = trows.at[slot], crows.at[slot], nrows.at[slot]
      rows = lane
      rowsK = lane * K
      zf = jnp.zeros((L,), f32)

      @plsc.parallel_loop(0, D, unroll=4, carry=(zf, *([zf] * K)))
      def out(dd, carry):
        accp = carry[0]
        accn = list(carry[1:])
        dvu = jnp.broadcast_to(dd, (L,)).astype(i32)
        t = plsc.load_gather(tro, [rows, dvu])
        cv = plsc.load_gather(cro, [rows, dvu])
        accp = accp + t * cv
        for k in range(K):
          accn[k] = accn[k] + t * plsc.load_gather(nro, [rowsK + k, dvu])
        return (accp, *accn)

      accp = out[0]
      accn = out[1:]
      posb[pl.ds(c * BC, L)] = accp
      for k in range(K):
        negb[pl.ds(c * BCK + k * BC, L)] = accn[k]

    fire(0, 0, semA)

    def pair(p, _):
      ca = 2 * p
      fire(ca + 1, 1, semB)
      drain(0, semA)
      compute(ca, 0)

      @pl.when(p < NPAIR - 1)
      def _():
        fire(ca + 2, 0, semA)

      drain(1, semB)
      compute(ca + 1, 1)
      return 0

    lax.fori_loop(0, NPAIR, pair, 0)
    pltpu.sync_copy(posb, pos_hbm.at[pl.ds(base, BPW)])
    pltpu.sync_copy(negb, nego_hbm.at[pl.ds(base * K, BPW * K)])

  return sc_kernel(target_w, context_w, neg_w_flat, emb_input_p, emb_output_p)


def _tc_loss(pos2, neg2):
  f32 = jnp.float32

  def tc_body(pos_ref, neg_ref, out_ref):
    p = pos_ref[...]
    n = neg_ref[...]

    def sp(x):
      return jnp.maximum(x, 0.0) + jnp.log1p(jnp.exp(-jnp.abs(x)))

    out_ref[0, 0] = (jnp.sum(sp(-p)) + jnp.sum(sp(n))) / B

  return pl.pallas_call(
      tc_body,
      out_shape=jax.ShapeDtypeStruct((1, 1), f32),
      out_specs=pl.BlockSpec(memory_space=pltpu.SMEM),
  )(pos2, neg2)


def kernel(target_w, context_w, neg_w, emb_input, emb_output):
  neg_w_flat = neg_w.astype(jnp.int32).reshape(B * K)
  tail_in = emb_input[NBLK * DP:, :]
  tail_out = emb_output[NBLK * DP:, :]
  emb_input_p, emb_output_p = _sc_transpose(
      emb_input.T, emb_output.T, tail_in, tail_out)
  pos, negs = _sc_scores(target_w.astype(jnp.int32),
                         context_w.astype(jnp.int32),
                         neg_w_flat, emb_input_p, emb_output_p)
  loss = _tc_loss(pos.reshape(B // 128, 128), negs.reshape(B * K // 128, 128))
  return loss[0, 0]
